# speedup vs baseline: 6.5497x; 6.5497x over previous
import functools

import jax
import jax.numpy as jnp
from jax import lax
from jax.experimental import pallas as pl
from jax.experimental.pallas import tpu as pltpu
from jax.experimental.pallas import tpu_sc as plsc

B = 1024
D = 128
N = 100000
TEMP_INV = 20.0
EPS = 0.1
NBLK = 2048
GRID = (N + NBLK - 1) // NBLK
NEG = -1e30


def _gather_rows_sc(features, idx):
    info = plsc.get_sparse_core_info()
    nw = info.num_cores * info.num_subcores
    bpw = B // nw
    mesh = plsc.VectorSubcoreMesh(core_axis_name="c", subcore_axis_name="s")

    @functools.partial(
        pl.kernel, mesh=mesh,
        out_type=jax.ShapeDtypeStruct((B, D), jnp.float32),
        scratch_types=[
            pltpu.VMEM((bpw,), jnp.int32),
            pltpu.VMEM((bpw, D), jnp.float32),
            pltpu.SemaphoreType.DMA,
        ],
    )
    def k(table_hbm, idx_hbm, out_hbm, idx_v, rows_v, sem):
        wid = lax.axis_index("s") * info.num_cores + lax.axis_index("c")
        base = wid * bpw
        pltpu.sync_copy(idx_hbm.at[pl.ds(base, bpw)], idx_v)
        pltpu.async_copy(table_hbm.at[idx_v], rows_v, sem).wait()
        pltpu.sync_copy(rows_v, out_hbm.at[pl.ds(base, bpw)])

    return k(features, idx)


def _tc_body(x_ref, f_ref, g_ref, out_ref, m_ref, s_ref, ss_ref):
    i = pl.program_id(0)

    @pl.when(i == 0)
    def _():
        m_ref[...] = jnp.full((B, 1), NEG, jnp.float32)
        s_ref[...] = jnp.zeros((B, 1), jnp.float32)
        ss_ref[...] = jnp.zeros((B, 1), jnp.float32)

    x = x_ref[...]
    nrm = jnp.sqrt(jnp.sum(x * x, axis=1, keepdims=True))
    xn = x / jnp.maximum(nrm, 1e-12)

    f = f_ref[...]
    logits = TEMP_INV * lax.dot_general(
        xn, f, (((1,), (1,)), ((), ())), preferred_element_type=jnp.float32)

    col = i * NBLK + lax.broadcasted_iota(jnp.int32, (1, NBLK), 1)
    valid = col < N
    lm = jnp.where(valid, logits, NEG)

    m_old = m_ref[...]
    m_new = jnp.maximum(m_old, jnp.max(lm, axis=1, keepdims=True))
    p = jnp.exp(lm - m_new)
    s_ref[...] = s_ref[...] * jnp.exp(m_old - m_new) + jnp.sum(
        p, axis=1, keepdims=True)
    m_ref[...] = m_new
    ss_ref[...] = ss_ref[...] + jnp.sum(
        jnp.where(valid, logits, 0.0), axis=1, keepdims=True)

    @pl.when(i == GRID - 1)
    def _():
        tl = TEMP_INV * jnp.sum(xn * g_ref[...], axis=1, keepdims=True)
        lse = m_ref[...] + jnp.log(s_ref[...])
        per_row = lse - (1.0 - EPS) * tl - (EPS / N) * ss_ref[...]
        out_ref[0, 0] = jnp.sum(per_row) / B


def _loss_tc(x, features, gathered, interpret=False):
    out = pl.pallas_call(
        _tc_body,
        grid=(GRID,),
        in_specs=[
            pl.BlockSpec((B, D), lambda i: (0, 0)),
            pl.BlockSpec((NBLK, D), lambda i: (i, 0)),
            pl.BlockSpec((B, D), lambda i: (0, 0)),
        ],
        out_specs=pl.BlockSpec(memory_space=pltpu.SMEM),
        out_shape=jax.ShapeDtypeStruct((1, 1), jnp.float32),
        scratch_shapes=[
            pltpu.VMEM((B, 1), jnp.float32),
            pltpu.VMEM((B, 1), jnp.float32),
            pltpu.VMEM((B, 1), jnp.float32),
        ],
        compiler_params=pltpu.CompilerParams(
            dimension_semantics=("arbitrary",)),
        interpret=interpret,
    )(x, features, gathered)
    return out[0, 0]


def kernel(inputs, targets, features):
    gathered = _gather_rows_sc(features, targets.astype(jnp.int32))
    return _loss_tc(inputs, features, gathered)


# speedup vs baseline: 7.7074x; 1.1768x over previous
import functools

import jax
import jax.numpy as jnp
from jax import lax
from jax.experimental import pallas as pl
from jax.experimental.pallas import tpu as pltpu
from jax.experimental.pallas import tpu_sc as plsc

B = 1024
D = 128
N = 100000
TEMP_INV = 20.0
EPS = 0.1
NBLK = 2000
GRID = N // NBLK
NEG = -1e30


def _gather_rows_sc(features, idx):
    info = plsc.get_sparse_core_info()
    nw = info.num_cores * info.num_subcores
    bpw = B // nw
    mesh = plsc.VectorSubcoreMesh(core_axis_name="c", subcore_axis_name="s")

    @functools.partial(
        pl.kernel, mesh=mesh,
        out_type=jax.ShapeDtypeStruct((B, D), jnp.float32),
        scratch_types=[
            pltpu.VMEM((bpw,), jnp.int32),
            pltpu.VMEM((bpw, D), jnp.float32),
            pltpu.SemaphoreType.DMA,
        ],
    )
    def k(table_hbm, idx_hbm, out_hbm, idx_v, rows_v, sem):
        wid = lax.axis_index("s") * info.num_cores + lax.axis_index("c")
        base = wid * bpw
        pltpu.sync_copy(idx_hbm.at[pl.ds(base, bpw)], idx_v)
        pltpu.async_copy(table_hbm.at[idx_v], rows_v, sem).wait()
        pltpu.sync_copy(rows_v, out_hbm.at[pl.ds(base, bpw)])

    return k(features, idx)


def _tc_body(x_ref, f_ref, g_ref, out_ref, m_ref, s_ref, cs_ref):
    i = pl.program_id(0)

    @pl.when(i == 0)
    def _():
        m_ref[...] = jnp.full((B, 1), NEG, jnp.float32)
        s_ref[...] = jnp.zeros((B, 1), jnp.float32)
        cs_ref[...] = jnp.zeros((1, D), jnp.float32)

    x = x_ref[...]
    nrm = jnp.sqrt(jnp.sum(x * x, axis=1, keepdims=True))
    xn = (TEMP_INV / jnp.maximum(nrm, 1e-12)) * x

    f = f_ref[...]
    logits = lax.dot_general(
        xn, f, (((1,), (1,)), ((), ())), preferred_element_type=jnp.float32)

    m_old = m_ref[...]
    m_new = jnp.maximum(m_old, jnp.max(logits, axis=1, keepdims=True))
    s_ref[...] = s_ref[...] * jnp.exp(m_old - m_new) + jnp.sum(
        jnp.exp(logits - m_new), axis=1, keepdims=True)
    m_ref[...] = m_new
    cs_ref[...] = cs_ref[...] + jnp.sum(f, axis=0, keepdims=True)

    @pl.when(i == GRID - 1)
    def _():
        tl = jnp.sum(xn * g_ref[...], axis=1, keepdims=True)
        ss = jnp.sum(xn * cs_ref[...], axis=1, keepdims=True)
        lse = m_ref[...] + jnp.log(s_ref[...])
        per_row = lse - (1.0 - EPS) * tl - (EPS / N) * ss
        out_ref[0, 0] = jnp.sum(per_row) / B


def _loss_tc(x, features, gathered, interpret=False):
    out = pl.pallas_call(
        _tc_body,
        grid=(GRID,),
        in_specs=[
            pl.BlockSpec((B, D), lambda i: (0, 0)),
            pl.BlockSpec((NBLK, D), lambda i: (i, 0)),
            pl.BlockSpec((B, D), lambda i: (0, 0)),
        ],
        out_specs=pl.BlockSpec(memory_space=pltpu.SMEM),
        out_shape=jax.ShapeDtypeStruct((1, 1), jnp.float32),
        scratch_shapes=[
            pltpu.VMEM((B, 1), jnp.float32),
            pltpu.VMEM((B, 1), jnp.float32),
            pltpu.VMEM((1, D), jnp.float32),
        ],
        compiler_params=pltpu.CompilerParams(
            dimension_semantics=("arbitrary",)),
        interpret=interpret,
    )(x, features, gathered)
    return out[0, 0]


def kernel(inputs, targets, features):
    gathered = _gather_rows_sc(features, targets.astype(jnp.int32))
    return _loss_tc(inputs, features, gathered)
